# baseline (speedup 1.0000x reference)
import functools

import jax
import jax.numpy as jnp
from jax import lax
from jax.experimental import pallas as pl
from jax.experimental.pallas import tpu as pltpu
from jax.experimental.pallas import tpu_sc as plsc

B, C, NUM, NCAT, V, D = 4096, 4, 13, 26, 100000, 32
N = B * C * NCAT
NC, NS = 2, 16
NW = NC * NS
ROWS_W = N // NW
IDX_ROWS = ROWS_W // 128
STEPS = 4
CHUNK = ROWS_W // STEPS
CHUNK_IDX_ROWS = IDX_ROWS // STEPS


def _sc_gather_body(s_cat_hbm, tables_hbm, out_hbm, idx_v, rows_v, sem):
    wid = lax.axis_index("s") * NC + lax.axis_index("c")
    base = wid * ROWS_W

    pltpu.sync_copy(s_cat_hbm.at[wid], idx_v)

    def add_offsets(r, _):
        n0 = base + r * 128
        for c8 in range(8):
            n = n0 + c8 * 16 + lax.iota(jnp.int32, 16)
            off = (n % NCAT) * V
            sl = pl.ds(r * 128 + c8 * 16, 16)
            idx_v[sl] = idx_v[sl] + off
        return _

    lax.fori_loop(0, IDX_ROWS, add_offsets, None)

    for step in range(STEPS):
        idx_slice = idx_v.at[pl.ds(step * CHUNK, CHUNK)]
        pltpu.async_copy(tables_hbm.at[idx_slice], rows_v, sem).wait()
        pltpu.sync_copy(rows_v, out_hbm.at[pl.ds(base + step * CHUNK, CHUNK)])


@jax.jit
def _sc_gather(s_cat_blocks, tables_flat):
    mesh = plsc.VectorSubcoreMesh(core_axis_name="c", subcore_axis_name="s")
    return pl.kernel(
        _sc_gather_body,
        out_type=jax.ShapeDtypeStruct((N, D), jnp.float32),
        mesh=mesh,
        scratch_types=[
            pltpu.VMEM((ROWS_W,), jnp.int32),
            pltpu.VMEM((CHUNK, D), jnp.float32),
            pltpu.SemaphoreType.DMA,
        ],
        compiler_params=pltpu.CompilerParams(use_tc_tiling_on_sc=False),
    )(s_cat_blocks, tables_flat)


BB = 128


def _assemble_body(s_cont_ref, w_ref, nan_ref, cat_ref, out_ref):
    sc = s_cont_ref[...]
    ne = sc[..., None] * w_ref[...][None, None]
    nan_mask = jnp.isnan(ne).all(axis=-1, keepdims=True)
    enc_cont = jnp.where(nan_mask, nan_ref[...][None, None], ne)
    cat = cat_ref[...].reshape(BB, C, NCAT, D)
    out_ref[...] = jnp.concatenate([enc_cont, cat], axis=2)


@jax.jit
def _assemble(s_cont, num_W, nan_embs, cat_rows):
    cat3 = cat_rows.reshape(B, C * NCAT, D)
    return pl.pallas_call(
        _assemble_body,
        grid=(B // BB,),
        in_specs=[
            pl.BlockSpec((BB, C, NUM), lambda i: (i, 0, 0)),
            pl.BlockSpec((NUM, D), lambda i: (0, 0)),
            pl.BlockSpec((NUM, D), lambda i: (0, 0)),
            pl.BlockSpec((BB, C * NCAT, D), lambda i: (i, 0, 0)),
        ],
        out_specs=pl.BlockSpec((BB, C, 39, D), lambda i: (i, 0, 0, 0)),
        out_shape=jax.ShapeDtypeStruct((B, C, NUM + NCAT, D), jnp.float32),
    )(s_cont, num_W, nan_embs, cat3)


def kernel(s_cont, s_cat, cat_tables, num_W, nan_embs):
    s_cat_blocks = s_cat.reshape(NW, ROWS_W)
    tables_flat = cat_tables.reshape(NCAT * V, D)
    cat_rows = _sc_gather(s_cat_blocks, tables_flat)
    return _assemble(s_cont, num_W, nan_embs, cat_rows)
